# baseline (speedup 1.0000x reference)
import functools

import jax
import jax.numpy as jnp
from jax import lax
from jax.experimental import pallas as pl
from jax.experimental.pallas import tpu as pltpu
from jax.experimental.pallas import tpu_sc as plsc

_LANES = 16
_CHUNK = 128


@functools.partial(jax.jit, static_argnames=("batch", "dim", "n_rows"))
def _run(user_v, item_v, users2d, items2d, *, batch, dim, n_rows):
    info = plsc.get_sparse_core_info()
    n_workers = info.num_cores * info.num_subcores
    b_per_w = batch // n_workers
    n_chunks = b_per_w // _CHUNK
    n_vecs = _CHUNK // _LANES
    runs_per_dim = n_rows // _LANES
    n_fetch = dim * _CHUNK

    mesh = plsc.VectorSubcoreMesh(core_axis_name="c", subcore_axis_name="s")

    @functools.partial(
        pl.kernel,
        out_type=jax.ShapeDtypeStruct((batch,), jnp.float32),
        mesh=mesh,
        scratch_types=[
            pltpu.VMEM((n_chunks, _CHUNK), jnp.int32),
            pltpu.VMEM((n_chunks, _CHUNK), jnp.int32),
            pltpu.VMEM((n_fetch,), jnp.int32),
            pltpu.VMEM((n_fetch,), jnp.int32),
            pltpu.VMEM((_CHUNK,), jnp.int32),
            pltpu.VMEM((_CHUNK,), jnp.int32),
            pltpu.VMEM((n_fetch, _LANES), jnp.float32),
            pltpu.VMEM((n_fetch, _LANES), jnp.float32),
            pltpu.VMEM((b_per_w,), jnp.float32),
            pltpu.SemaphoreType.DMA,
            pltpu.SemaphoreType.DMA,
        ],
        compiler_params=pltpu.CompilerParams(
            needs_layout_passes=False, use_tc_tiling_on_sc=False),
    )
    def sc_kernel(ut_hbm, it_hbm, users_hbm, items_hbm, out_hbm,
                  uidx_v, iidx_v, urid_v, irid_v, ucol_v, icol_v,
                  urows_v, irows_v, scores_v, usem, isem):
        wid = lax.axis_index("s") * info.num_cores + lax.axis_index("c")
        idx_row0 = wid * n_chunks

        pltpu.sync_copy(users_hbm.at[pl.ds(idx_row0, n_chunks)], uidx_v)
        pltpu.sync_copy(items_hbm.at[pl.ds(idx_row0, n_chunks)], iidx_v)

        lanes = lax.iota(jnp.int32, _LANES)

        for c in range(n_chunks):
            for v in range(n_vecs):
                sl = pl.ds(v * _LANES, _LANES)
                ub = uidx_v[c, sl]
                ib = iidx_v[c, sl]
                ucol_v[sl] = lax.bitwise_and(ub, _LANES - 1)
                icol_v[sl] = lax.bitwise_and(ib, _LANES - 1)
                urun = lax.shift_right_logical(ub, 4)
                irun = lax.shift_right_logical(ib, 4)
                for d in range(dim):
                    dst = pl.ds(d * _CHUNK + v * _LANES, _LANES)
                    urid_v[dst] = urun + d * runs_per_dim
                    irid_v[dst] = irun + d * runs_per_dim

            cu = pltpu.async_copy(ut_hbm.at[urid_v], urows_v, usem)
            ci = pltpu.async_copy(it_hbm.at[irid_v], irows_v, isem)
            cu.wait()
            ci.wait()

            for g in range(n_vecs):
                gsl = pl.ds(g * _LANES, _LANES)
                ucol = ucol_v[gsl]
                icol = icol_v[gsl]
                rbase = lanes + g * _LANES
                acc = jnp.zeros((_LANES,), jnp.float32)
                for d in range(dim):
                    rid = rbase + d * _CHUNK
                    uu = plsc.load_gather(urows_v, [rid, ucol])
                    ii = plsc.load_gather(irows_v, [rid, icol])
                    acc = acc + uu * ii
                scores_v[pl.ds(c * _CHUNK + g * _LANES, _LANES)] = acc

        pltpu.sync_copy(scores_v, out_hbm.at[pl.ds(wid * b_per_w, b_per_w)])

    return sc_kernel(user_v, item_v, users2d, items2d)


def kernel(user_table, item_table, users, items):
    batch = users.shape[0]
    n_rows, dim = user_table.shape
    users2d = users.astype(jnp.int32).reshape(batch // _CHUNK, _CHUNK)
    items2d = items.astype(jnp.int32).reshape(batch // _CHUNK, _CHUNK)
    user_v = user_table.T.reshape(n_rows, dim)
    item_v = item_table.T.reshape(n_rows, dim)
    return _run(user_v, item_v, users2d, items2d,
                batch=batch, dim=dim, n_rows=n_rows)


# speedup vs baseline: 1.0007x; 1.0007x over previous
import functools

import jax
import jax.numpy as jnp
from jax import lax
from jax.experimental import pallas as pl
from jax.experimental.pallas import tpu as pltpu
from jax.experimental.pallas import tpu_sc as plsc

_LANES = 16
_CHUNK = 128


@functools.partial(jax.jit, static_argnames=("batch", "dim", "n_rows"))
def _run(user_t, item_t, users2d, items2d, *, batch, dim, n_rows):
    info = plsc.get_sparse_core_info()
    n_workers = info.num_cores * info.num_subcores
    b_per_w = batch // n_workers
    n_chunks = b_per_w // _CHUNK
    n_vecs = _CHUNK // _LANES

    mesh = plsc.VectorSubcoreMesh(core_axis_name="c", subcore_axis_name="s")

    @functools.partial(
        pl.kernel,
        out_type=jax.ShapeDtypeStruct((batch,), jnp.float32),
        mesh=mesh,
        scratch_types=[
            pltpu.SMEM((n_chunks, _CHUNK), jnp.int32),
            pltpu.SMEM((n_chunks, _CHUNK), jnp.int32),
            pltpu.VMEM((n_chunks, _CHUNK), jnp.int32),
            pltpu.VMEM((n_chunks, _CHUNK), jnp.int32),
            pltpu.VMEM((_CHUNK * _LANES, _LANES), jnp.float32),
            pltpu.VMEM((_CHUNK * _LANES, _LANES), jnp.float32),
            pltpu.VMEM((b_per_w,), jnp.float32),
            pltpu.VMEM_SHARED((info.num_subcores, n_chunks, _CHUNK),
                              jnp.int32),
            pltpu.VMEM_SHARED((info.num_subcores, n_chunks, _CHUNK),
                              jnp.int32),
            pltpu.SemaphoreType.DMA,
            pltpu.SemaphoreType.DMA,
        ],
        compiler_params=pltpu.CompilerParams(
            needs_layout_passes=False, use_tc_tiling_on_sc=False),
    )
    def sc_kernel(ut_hbm, it_hbm, users_hbm, items_hbm, out_hbm,
                  uidx_s, iidx_s, uidx_v, iidx_v, urows_v, irows_v,
                  scores_v, ushr_v, ishr_v, usem, isem):
        sid = lax.axis_index("s")
        wid = sid * info.num_cores + lax.axis_index("c")
        idx_row0 = wid * n_chunks

        pltpu.sync_copy(users_hbm.at[pl.ds(idx_row0, n_chunks)], uidx_v)
        pltpu.sync_copy(items_hbm.at[pl.ds(idx_row0, n_chunks)], iidx_v)
        pltpu.sync_copy(uidx_v, ushr_v.at[sid])
        pltpu.sync_copy(iidx_v, ishr_v.at[sid])
        pltpu.sync_copy(ushr_v.at[sid], uidx_s)
        pltpu.sync_copy(ishr_v.at[sid], iidx_s)

        lanes = lax.iota(jnp.int32, _LANES)

        for c in range(n_chunks):
            def fire_body(e, carry):
                bu = uidx_s[c, e]
                bi = iidx_s[c, e]
                urun = pl.multiple_of((bu // _LANES) * _LANES, _LANES)
                irun = pl.multiple_of((bi // _LANES) * _LANES, _LANES)
                dst = pl.ds(pl.multiple_of(e * _LANES, _LANES), _LANES)
                pltpu.async_copy(
                    ut_hbm.at[:, pl.ds(urun, _LANES)], urows_v.at[dst, :],
                    usem)
                pltpu.async_copy(
                    it_hbm.at[:, pl.ds(irun, _LANES)], irows_v.at[dst, :],
                    isem)
                return carry

            lax.fori_loop(0, _CHUNK, fire_body, 0)

            def drain_body(e, carry):
                dst = pl.ds(pl.multiple_of(e * _LANES, _LANES), _LANES)
                pltpu.make_async_copy(
                    ut_hbm.at[:, pl.ds(0, _LANES)], urows_v.at[dst, :],
                    usem).wait()
                pltpu.make_async_copy(
                    it_hbm.at[:, pl.ds(0, _LANES)], irows_v.at[dst, :],
                    isem).wait()
                return carry

            lax.fori_loop(0, _CHUNK, drain_body, 0)

            for g in range(n_vecs):
                gsl = pl.ds(g * _LANES, _LANES)
                ucol = lax.bitwise_and(uidx_v[c, gsl], _LANES - 1)
                icol = lax.bitwise_and(iidx_v[c, gsl], _LANES - 1)
                rbase = (lanes + g * _LANES) * _LANES
                acc = jnp.zeros((_LANES,), jnp.float32)
                for d in range(dim):
                    rid = rbase + d
                    uu = plsc.load_gather(urows_v, [rid, ucol])
                    ii = plsc.load_gather(irows_v, [rid, icol])
                    acc = acc + uu * ii
                scores_v[pl.ds(c * _CHUNK + g * _LANES, _LANES)] = acc

        pltpu.sync_copy(scores_v, out_hbm.at[pl.ds(wid * b_per_w, b_per_w)])

    return sc_kernel(user_t, item_t, users2d, items2d)


def kernel(user_table, item_table, users, items):
    batch = users.shape[0]
    n_rows, dim = user_table.shape
    users2d = users.astype(jnp.int32).reshape(batch // _CHUNK, _CHUNK)
    items2d = items.astype(jnp.int32).reshape(batch // _CHUNK, _CHUNK)
    return _run(user_table.T, item_table.T, users2d, items2d,
                batch=batch, dim=dim, n_rows=n_rows)
